# baseline (speedup 1.0000x reference)
import functools

import jax
import jax.numpy as jnp
from jax import lax
from jax.experimental import pallas as pl
from jax.experimental.pallas import tpu as pltpu
from jax.experimental.pallas import tpu_sc as plsc


def _sc_info():
    try:
        info = plsc.get_sparse_core_info()
        return info.num_cores, info.num_subcores
    except Exception:
        return 2, 16


def _sc_gather(table, flat_idx):
    V, D = table.shape
    T = flat_idx.shape[0]
    NC, NS = _sc_info()
    NW = NC * NS
    assert T % NW == 0
    per_w = T // NW
    C = 128
    while per_w % (2 * C):
        C -= 8
    nchunks = per_w // C

    mesh = plsc.VectorSubcoreMesh(core_axis_name="c", subcore_axis_name="s",
                                  num_cores=NC, num_subcores=NS)

    @functools.partial(
        pl.kernel,
        mesh=mesh,
        out_type=jax.ShapeDtypeStruct((T, D), table.dtype),
        scratch_types=[
            pltpu.VMEM((per_w,), jnp.int32),
            pltpu.VMEM((C, D), table.dtype),
            pltpu.VMEM((C, D), table.dtype),
            pltpu.SemaphoreType.DMA,
            pltpu.SemaphoreType.DMA,
            pltpu.SemaphoreType.DMA,
            pltpu.SemaphoreType.DMA,
        ],
    )
    def gather_k(table_hbm, idx_hbm, out_hbm, idx_v, rows0, rows1,
                 sg0, sg1, so0, so1):
        wid = lax.axis_index("s") * NC + lax.axis_index("c")
        base = wid * per_w
        pltpu.sync_copy(idx_hbm.at[pl.ds(base, per_w)], idx_v)

        def g_copy(j, rows, sg):
            return pltpu.make_async_copy(
                table_hbm.at[idx_v.at[pl.ds(j * C, C)]], rows, sg)

        def o_copy(j, rows, so):
            return pltpu.make_async_copy(
                rows, out_hbm.at[pl.ds(base + j * C, C)], so)

        g_copy(0, rows0, sg0).start()
        g_copy(1, rows1, sg1).start()

        def pair(g, carry):
            j0 = g * 2
            g_copy(j0, rows0, sg0).wait()
            o_copy(j0, rows0, so0).start()
            g_copy(j0 + 1, rows1, sg1).wait()
            o_copy(j0 + 1, rows1, so1).start()
            o_copy(j0, rows0, so0).wait()
            g_copy(j0 + 2, rows0, sg0).start()
            o_copy(j0 + 1, rows1, so1).wait()
            g_copy(j0 + 3, rows1, sg1).start()
            return carry

        lax.fori_loop(0, nchunks // 2 - 1, pair, 0)
        jl = nchunks - 2
        g_copy(jl, rows0, sg0).wait()
        o_copy(jl, rows0, so0).start()
        g_copy(jl + 1, rows1, sg1).wait()
        o_copy(jl + 1, rows1, so1).start()
        o_copy(jl, rows0, so0).wait()
        o_copy(jl + 1, rows1, so1).wait()

    return gather_k(table, flat_idx)


def _lstm_layer(m_sm, h_in, wcat, bias, wout, b, relu):
    DEG, N, D = m_sm.shape
    dout = wout.shape[1]
    B = 400 if N % 400 == 0 else 320
    assert N % B == 0

    cdt = jnp.bfloat16

    def body(m_ref, h_ref, wcat_ref, bias_ref, wout_ref, b_ref, out_ref):
        hin = h_ref[...].astype(cdt)
        wc = wcat_ref[...]
        bs = bias_ref[...]

        def sg(x):
            return 0.5 + 0.5 * jnp.tanh(0.5 * x)

        def cell(x, h, c):
            xh = jnp.concatenate([x, h.astype(cdt)], axis=1)
            gates = jnp.dot(xh, wc, preferred_element_type=jnp.float32) + bs
            i = sg(gates[:, :D])
            f = sg(gates[:, D:2 * D])
            g = jnp.tanh(gates[:, 2 * D:3 * D])
            o = sg(gates[:, 3 * D:])
            c = f * c + i * g
            h = o * jnp.tanh(c)
            return h, c

        z = jnp.zeros((B, D), jnp.float32)
        h, c = z, z
        for t in range(DEG):
            h, c = cell(m_ref[t].astype(cdt), h, c)
        hT = h
        hcat = jnp.concatenate([hin, hT.astype(cdt)], axis=1)
        rst = jnp.dot(hcat, wout_ref[...], preferred_element_type=jnp.float32)
        rst = rst + b_ref[...]
        if relu:
            rst = jnp.maximum(rst, 0.0)
        out_ref[...] = rst

    return pl.pallas_call(
        body,
        grid=(N // B,),
        in_specs=[
            pl.BlockSpec((DEG, B, D), lambda i: (0, i, 0)),
            pl.BlockSpec((B, D), lambda i: (i, 0)),
            pl.BlockSpec((2 * D, 4 * D), lambda i: (0, 0)),
            pl.BlockSpec((1, 4 * D), lambda i: (0, 0)),
            pl.BlockSpec((2 * D, dout), lambda i: (0, 0)),
            pl.BlockSpec((1, dout), lambda i: (0, 0)),
        ],
        out_specs=pl.BlockSpec((B, dout), lambda i: (i, 0)),
        out_shape=jax.ShapeDtypeStruct((N, dout), jnp.float32),
    )(m_sm, h_in, wcat, bias, wout, b)


def kernel(p, neigh_idx, params):
    N, P = p.shape
    DEG = neigh_idx.shape[1]
    S = 1
    BS = 2048 * S
    NP_ = ((N + BS - 1) // BS) * BS
    Ns = NP_ // S
    deg_col = jnp.full((N, 1), jnp.float32(DEG) / jnp.float32(N), jnp.float32)
    h = jnp.concatenate([deg_col, p.astype(jnp.float32)], axis=1)
    h = jnp.pad(h, ((0, NP_ - N), (0, 0)))
    idx_pad = jnp.pad(neigh_idx.astype(jnp.int32), ((0, NP_ - N), (0, 0)))
    shard_idx = [idx_pad[s * Ns:(s + 1) * Ns].T.reshape(-1) for s in range(S)]

    n_layers = len(params['layers'])
    for li, lp in enumerate(params['layers']):
        D = h.shape[1]
        wcat = jnp.concatenate([lp['Wih'].T, lp['Whh'].T],
                               axis=0).astype(jnp.bfloat16)
        bias = (lp['bih'] + lp['bhh'])[None, :]
        wout = jnp.concatenate([lp['W_self'].T, lp['W_neigh'].T],
                               axis=0).astype(jnp.bfloat16)
        b = lp['b'][None, :]
        relu = li < n_layers - 1
        rows = [_sc_gather(h, shard_idx[s]) for s in range(S)]
        outs = [
            _lstm_layer(rows[s].reshape(DEG, Ns, D),
                        h[s * Ns:(s + 1) * Ns], wcat, bias, wout, b, relu)
            for s in range(S)
        ]
        h = jnp.concatenate(outs, axis=0)
    return h[:N]


# speedup vs baseline: 1.0414x; 1.0414x over previous
import functools

import jax
import jax.numpy as jnp
from jax import lax
from jax.experimental import pallas as pl
from jax.experimental.pallas import tpu as pltpu
from jax.experimental.pallas import tpu_sc as plsc


def _sc_info():
    try:
        info = plsc.get_sparse_core_info()
        return info.num_cores, info.num_subcores
    except Exception:
        return 2, 16


def _sc_gather(table, flat_idx):
    V, D = table.shape
    T = flat_idx.shape[0]
    NC, NS = _sc_info()
    NW = NC * NS
    assert T % NW == 0
    per_w = T // NW
    C = 40
    while per_w % (2 * C):
        C -= 8
    nchunks = per_w // C

    mesh = plsc.VectorSubcoreMesh(core_axis_name="c", subcore_axis_name="s",
                                  num_cores=NC, num_subcores=NS)

    @functools.partial(
        pl.kernel,
        mesh=mesh,
        out_type=jax.ShapeDtypeStruct((T, D), table.dtype),
        scratch_types=[
            pltpu.VMEM((per_w,), jnp.int32),
            pltpu.VMEM((C, D), table.dtype),
            pltpu.VMEM((C, D), table.dtype),
            pltpu.SemaphoreType.DMA,
            pltpu.SemaphoreType.DMA,
            pltpu.SemaphoreType.DMA,
            pltpu.SemaphoreType.DMA,
        ],
    )
    def gather_k(table_hbm, idx_hbm, out_hbm, idx_v, rows0, rows1,
                 sg0, sg1, so0, so1):
        wid = lax.axis_index("s") * NC + lax.axis_index("c")
        base = wid * per_w
        pltpu.sync_copy(idx_hbm.at[pl.ds(base, per_w)], idx_v)

        def g_copy(j, rows, sg):
            return pltpu.make_async_copy(
                table_hbm.at[idx_v.at[pl.ds(j * C, C)]], rows, sg)

        def o_copy(j, rows, so):
            return pltpu.make_async_copy(
                rows, out_hbm.at[pl.ds(base + j * C, C)], so)

        g_copy(0, rows0, sg0).start()
        g_copy(1, rows1, sg1).start()

        def pair(g, carry):
            j0 = g * 2
            g_copy(j0, rows0, sg0).wait()
            o_copy(j0, rows0, so0).start()
            g_copy(j0 + 1, rows1, sg1).wait()
            o_copy(j0 + 1, rows1, so1).start()
            o_copy(j0, rows0, so0).wait()
            g_copy(j0 + 2, rows0, sg0).start()
            o_copy(j0 + 1, rows1, so1).wait()
            g_copy(j0 + 3, rows1, sg1).start()
            return carry

        lax.fori_loop(0, nchunks // 2 - 1, pair, 0)
        jl = nchunks - 2
        g_copy(jl, rows0, sg0).wait()
        o_copy(jl, rows0, so0).start()
        g_copy(jl + 1, rows1, sg1).wait()
        o_copy(jl + 1, rows1, so1).start()
        o_copy(jl, rows0, so0).wait()
        o_copy(jl + 1, rows1, so1).wait()

    return gather_k(table, flat_idx)


def _lstm_layer(m_sm, h_in, wcat, bias, wout, b, relu):
    DEG, N, D = m_sm.shape
    dout = wout.shape[1]
    B = 400 if N % 400 == 0 else 512
    assert N % B == 0

    cdt = jnp.bfloat16

    def body(m_ref, h_ref, wcat_ref, bias_ref, wout_ref, b_ref, out_ref):
        hin = h_ref[...].astype(cdt)
        wc = wcat_ref[...]
        bs = bias_ref[...]

        def sg(x):
            return 0.5 + 0.5 * jnp.tanh(0.5 * x)

        def cell(x, h, c):
            xh = jnp.concatenate([x, h.astype(cdt)], axis=1)
            gates = jnp.dot(xh, wc, preferred_element_type=jnp.float32) + bs
            i = sg(gates[:, :D])
            f = sg(gates[:, D:2 * D])
            g = jnp.tanh(gates[:, 2 * D:3 * D])
            o = sg(gates[:, 3 * D:])
            c = f * c + i * g
            h = o * jnp.tanh(c)
            return h, c

        z = jnp.zeros((B, D), jnp.float32)
        h, c = z, z
        for t in range(DEG):
            h, c = cell(m_ref[t].astype(cdt), h, c)
        hT = h
        hcat = jnp.concatenate([hin, hT.astype(cdt)], axis=1)
        rst = jnp.dot(hcat, wout_ref[...], preferred_element_type=jnp.float32)
        rst = rst + b_ref[...]
        if relu:
            rst = jnp.maximum(rst, 0.0)
        out_ref[...] = rst

    return pl.pallas_call(
        body,
        grid=(N // B,),
        in_specs=[
            pl.BlockSpec((DEG, B, D), lambda i: (0, i, 0)),
            pl.BlockSpec((B, D), lambda i: (i, 0)),
            pl.BlockSpec((2 * D, 4 * D), lambda i: (0, 0)),
            pl.BlockSpec((1, 4 * D), lambda i: (0, 0)),
            pl.BlockSpec((2 * D, dout), lambda i: (0, 0)),
            pl.BlockSpec((1, dout), lambda i: (0, 0)),
        ],
        out_specs=pl.BlockSpec((B, dout), lambda i: (i, 0)),
        out_shape=jax.ShapeDtypeStruct((N, dout), jnp.float32),
    )(m_sm, h_in, wcat, bias, wout, b)


def kernel(p, neigh_idx, params):
    N, P = p.shape
    DEG = neigh_idx.shape[1]
    S = 1
    BS = 2048 * S
    NP_ = ((N + BS - 1) // BS) * BS
    Ns = NP_ // S
    deg_col = jnp.full((N, 1), jnp.float32(DEG) / jnp.float32(N), jnp.float32)
    h = jnp.concatenate([deg_col, p.astype(jnp.float32)], axis=1)
    h = jnp.pad(h, ((0, NP_ - N), (0, 0)))
    idx_pad = jnp.pad(neigh_idx.astype(jnp.int32), ((0, NP_ - N), (0, 0)))
    shard_idx = [idx_pad[s * Ns:(s + 1) * Ns].T.reshape(-1) for s in range(S)]

    n_layers = len(params['layers'])
    for li, lp in enumerate(params['layers']):
        D = h.shape[1]
        wcat = jnp.concatenate([lp['Wih'].T, lp['Whh'].T],
                               axis=0).astype(jnp.bfloat16)
        bias = (lp['bih'] + lp['bhh'])[None, :]
        wout = jnp.concatenate([lp['W_self'].T, lp['W_neigh'].T],
                               axis=0).astype(jnp.bfloat16)
        b = lp['b'][None, :]
        relu = li < n_layers - 1
        rows = [_sc_gather(h, shard_idx[s]) for s in range(S)]
        outs = [
            _lstm_layer(rows[s].reshape(DEG, Ns, D),
                        h[s * Ns:(s + 1) * Ns], wcat, bias, wout, b, relu)
            for s in range(S)
        ]
        h = jnp.concatenate(outs, axis=0)
    return h[:N]


# speedup vs baseline: 1.8794x; 1.8047x over previous
import functools

import jax
import jax.numpy as jnp
from jax import lax
from jax.experimental import pallas as pl
from jax.experimental.pallas import tpu as pltpu
from jax.experimental.pallas import tpu_sc as plsc


def _sc_info():
    try:
        info = plsc.get_sparse_core_info()
        return info.num_cores, info.num_subcores
    except Exception:
        return 2, 16


def _sc_gather(table, flat_idx):
    V, D = table.shape
    T = flat_idx.shape[0]
    NC, NS = _sc_info()
    NW = NC * NS
    assert T % NW == 0
    per_w = T // NW
    C = 40
    while per_w % (2 * C):
        C -= 8
    nchunks = per_w // C

    mesh = plsc.VectorSubcoreMesh(core_axis_name="c", subcore_axis_name="s",
                                  num_cores=NC, num_subcores=NS)

    @functools.partial(
        pl.kernel,
        mesh=mesh,
        out_type=jax.ShapeDtypeStruct((T, D), table.dtype),
        scratch_types=[
            pltpu.VMEM((per_w,), jnp.int32),
            pltpu.VMEM((C, D), table.dtype),
            pltpu.VMEM((C, D), table.dtype),
            pltpu.SemaphoreType.DMA,
            pltpu.SemaphoreType.DMA,
            pltpu.SemaphoreType.DMA,
            pltpu.SemaphoreType.DMA,
        ],
    )
    def gather_k(table_hbm, idx_hbm, out_hbm, idx_v, rows0, rows1,
                 sg0, sg1, so0, so1):
        wid = lax.axis_index("s") * NC + lax.axis_index("c")
        base = wid * per_w
        pltpu.sync_copy(idx_hbm.at[pl.ds(base, per_w)], idx_v)

        def g_copy(j, rows, sg):
            return pltpu.make_async_copy(
                table_hbm.at[idx_v.at[pl.ds(j * C, C)]], rows, sg)

        def o_copy(j, rows, so):
            return pltpu.make_async_copy(
                rows, out_hbm.at[pl.ds(base + j * C, C)], so)

        g_copy(0, rows0, sg0).start()
        g_copy(1, rows1, sg1).start()

        def pair(g, carry):
            j0 = g * 2
            g_copy(j0, rows0, sg0).wait()
            o_copy(j0, rows0, so0).start()
            g_copy(j0 + 1, rows1, sg1).wait()
            o_copy(j0 + 1, rows1, so1).start()
            o_copy(j0, rows0, so0).wait()
            g_copy(j0 + 2, rows0, sg0).start()
            o_copy(j0 + 1, rows1, so1).wait()
            g_copy(j0 + 3, rows1, sg1).start()
            return carry

        lax.fori_loop(0, nchunks // 2 - 1, pair, 0)
        jl = nchunks - 2
        g_copy(jl, rows0, sg0).wait()
        o_copy(jl, rows0, so0).start()
        g_copy(jl + 1, rows1, sg1).wait()
        o_copy(jl + 1, rows1, so1).start()
        o_copy(jl, rows0, so0).wait()
        o_copy(jl + 1, rows1, so1).wait()

    return gather_k(table, flat_idx)


def _lstm_layer(m_sm, h_in, wcat, bias, wout, b, relu):
    DEG, N, D = m_sm.shape
    dout = wout.shape[1]
    B = 400
    assert N % B == 0

    cdt = jnp.bfloat16

    def body(m_ref, h_ref, wcat_ref, bias_ref, wout_ref, b_ref, out_ref):
        hin = h_ref[...].astype(cdt)
        wc = wcat_ref[...]
        bs = bias_ref[...]

        def sg(x):
            return 0.5 + 0.5 * jnp.tanh(0.5 * x)

        def cell(x, h, c):
            xh = jnp.concatenate([x, h.astype(cdt)], axis=1)
            gates = jnp.dot(xh, wc, preferred_element_type=jnp.float32) + bs
            i = sg(gates[:, :D])
            f = sg(gates[:, D:2 * D])
            g = jnp.tanh(gates[:, 2 * D:3 * D])
            o = sg(gates[:, 3 * D:])
            c = f * c + i * g
            h = o * jnp.tanh(c)
            return h, c

        z = jnp.zeros((B, D), jnp.float32)
        h, c = z, z
        for t in range(DEG):
            h, c = cell(m_ref[t].astype(cdt), h, c)
        hT = h
        hcat = jnp.concatenate([hin, hT.astype(cdt)], axis=1)
        rst = jnp.dot(hcat, wout_ref[...], preferred_element_type=jnp.float32)
        rst = rst + b_ref[...]
        if relu:
            rst = jnp.maximum(rst, 0.0)
        out_ref[...] = rst

    return pl.pallas_call(
        body,
        grid=(N // B,),
        in_specs=[
            pl.BlockSpec((DEG, B, D), lambda i: (0, i, 0)),
            pl.BlockSpec((B, D), lambda i: (i, 0)),
            pl.BlockSpec((2 * D, 4 * D), lambda i: (0, 0)),
            pl.BlockSpec((1, 4 * D), lambda i: (0, 0)),
            pl.BlockSpec((2 * D, dout), lambda i: (0, 0)),
            pl.BlockSpec((1, dout), lambda i: (0, 0)),
        ],
        out_specs=pl.BlockSpec((B, dout), lambda i: (i, 0)),
        out_shape=jax.ShapeDtypeStruct((N, dout), jnp.float32),
    )(m_sm, h_in, wcat, bias, wout, b)


def kernel(p, neigh_idx, params):
    N, P = p.shape
    DEG = neigh_idx.shape[1]
    deg_col = jnp.full((N, 1), jnp.float32(DEG) / jnp.float32(N), jnp.float32)
    h = jnp.concatenate([deg_col, p.astype(jnp.float32)], axis=1)
    flat_idx = neigh_idx.T.reshape(-1).astype(jnp.int32)

    n_layers = len(params['layers'])
    for li, lp in enumerate(params['layers']):
        D = h.shape[1]
        wcat = jnp.concatenate([lp['Wih'].T, lp['Whh'].T],
                               axis=0).astype(jnp.bfloat16)
        bias = (lp['bih'] + lp['bhh'])[None, :]
        wout = jnp.concatenate([lp['W_self'].T, lp['W_neigh'].T],
                               axis=0).astype(jnp.bfloat16)
        b = lp['b'][None, :]
        rows = _sc_gather(h, flat_idx)
        m_sm = rows.reshape(DEG, N, D)
        h = _lstm_layer(m_sm, h, wcat, bias, wout, b,
                        relu=(li < n_layers - 1))
    return h
